# baseline (speedup 1.0000x reference)
import functools

import jax
import jax.numpy as jnp
import numpy as np
from jax import lax
from jax.experimental import pallas as pl
from jax.experimental.pallas import tpu as pltpu
from jax.experimental.pallas import tpu_sc as plsc

N = 10000
E = 320000
D = 128

NC = 2
NS = 16
NW = NC * NS
DH = D // NC
CH = 100
NCHA = 200
CHD = 50
NCHD = 200
NBUF = 5
DBUF = 4
RPT = 632
NP = NS * RPT
BLK = 1000
GRID = N // BLK

_f32 = jnp.float32
_mesh = plsc.VectorSubcoreMesh(core_axis_name="c", subcore_axis_name="s")


@functools.partial(
    pl.kernel,
    mesh=_mesh,
    compiler_params=pltpu.CompilerParams(use_tc_tiling_on_sc=False),
    out_type=jax.ShapeDtypeStruct((NC, NP, 16), _f32),
    scratch_types=[
        pltpu.VMEM_SHARED((NP, 16), _f32),
        pltpu.VMEM((CHD, 16), _f32),
        pltpu.VMEM((NCHD, CHD), jnp.int32),
        pltpu.SemaphoreType.DMA,
        pltpu.SemaphoreType.DMA,
        pltpu.SemaphoreType.DMA,
        pltpu.SemaphoreType.DMA,
    ],
)
def _deg_sc(dst_hbm, z_hbm, out_hbm, shared, ones_v, didx, s0, s1, s2, s3):
    c = lax.axis_index("c")
    s = lax.axis_index("s")
    ssems = [s0, s1, s2, s3]

    def fill_ones(i, carry):
        ones_v[i] = jnp.full((16,), 1.0, _f32)
        return carry
    lax.fori_loop(0, CHD, fill_ones, 0)

    pltpu.sync_copy(dst_hbm.at[c * NS + s], didx)
    pltpu.sync_copy(z_hbm.at[pl.ds(s * RPT, RPT)],
                    shared.at[pl.ds(s * RPT, RPT)])
    plsc.subcore_barrier()

    def group(g, carry):
        for b in range(DBUF):
            t = g * DBUF + b

            @pl.when(t >= DBUF)
            def _():
                pltpu.make_async_copy(ones_v, shared.at[didx.at[t - DBUF]],
                                      ssems[b]).wait()
            pltpu.async_copy(ones_v, shared.at[didx.at[t]], ssems[b], add=True)
        return carry
    lax.fori_loop(0, NCHD // DBUF, group, 0)
    for b in range(DBUF):
        pltpu.make_async_copy(ones_v, shared.at[didx.at[NCHD - DBUF + b]],
                              ssems[b]).wait()

    plsc.subcore_barrier()
    pltpu.sync_copy(shared.at[pl.ds(s * RPT, RPT)],
                    out_hbm.at[c, pl.ds(s * RPT, RPT)])


@functools.partial(
    pl.kernel,
    mesh=_mesh,
    compiler_params=pltpu.CompilerParams(use_tc_tiling_on_sc=False),
    out_type=jax.ShapeDtypeStruct((NC, NP, DH), _f32),
    scratch_types=[
        pltpu.VMEM_SHARED((NP, DH), _f32),
        pltpu.VMEM((CH, DH), _f32),
        pltpu.VMEM((CH, DH), _f32),
        pltpu.VMEM((CH, DH), _f32),
        pltpu.VMEM((CH, DH), _f32),
        pltpu.VMEM((CH, DH), _f32),
        pltpu.VMEM((NCHA, CH), jnp.int32),
        pltpu.VMEM((NCHA, CH), jnp.int32),
        pltpu.SemaphoreType.DMA,
        pltpu.SemaphoreType.DMA,
        pltpu.SemaphoreType.DMA,
        pltpu.SemaphoreType.DMA,
        pltpu.SemaphoreType.DMA,
        pltpu.SemaphoreType.DMA,
        pltpu.SemaphoreType.DMA,
        pltpu.SemaphoreType.DMA,
        pltpu.SemaphoreType.DMA,
        pltpu.SemaphoreType.DMA,
    ],
)
def _agg_sc(h_hbm, src_hbm, dst_hbm, z_hbm, out_hbm, shared,
            r0, r1, r2, r3, r4, sidx, didx,
            g0, g1, g2, g3, g4,
            t0, t1, t2, t3, t4):
    c = lax.axis_index("c")
    s = lax.axis_index("s")
    rows = [r0, r1, r2, r3, r4]
    gsems = [g0, g1, g2, g3, g4]
    ssems = [t0, t1, t2, t3, t4]
    hsrc = h_hbm.at[c]

    pltpu.sync_copy(src_hbm.at[s], sidx)
    pltpu.sync_copy(dst_hbm.at[s], didx)
    pltpu.sync_copy(z_hbm.at[pl.ds(s * RPT, RPT)],
                    shared.at[pl.ds(s * RPT, RPT)])
    plsc.subcore_barrier()

    for b in range(NBUF):
        pltpu.async_copy(hsrc.at[sidx.at[b]], rows[b], gsems[b])

    def group(g, carry):
        for b in range(NBUF):
            t = g * NBUF + b
            bp = (b - 1) % NBUF
            pltpu.make_async_copy(hsrc.at[sidx.at[t]], rows[b], gsems[b]).wait()
            pltpu.async_copy(rows[b], shared.at[didx.at[t]], ssems[b], add=True)

            @pl.when(jnp.logical_and(t >= 1, t + NBUF - 1 < NCHA))
            def _():
                pltpu.make_async_copy(rows[bp], shared.at[didx.at[t - 1]],
                                      ssems[bp]).wait()
                pltpu.async_copy(hsrc.at[sidx.at[t + NBUF - 1]], rows[bp],
                                 gsems[bp])
        return carry
    lax.fori_loop(0, NCHA // NBUF, group, 0)

    for b in range(NBUF):
        t = NCHA - NBUF + b
        pltpu.make_async_copy(rows[b], shared.at[didx.at[t]], ssems[b]).wait()

    plsc.subcore_barrier()
    pltpu.sync_copy(shared.at[pl.ds(s * RPT, RPT)],
                    out_hbm.at[c, pl.ds(s * RPT, RPT)])


def _tc_mm_body(x_ref, w_ref, h_ref):
    h_ref[...] = jnp.dot(x_ref[...], w_ref[...], preferred_element_type=_f32)


def _tc_mm(x, W1):
    return pl.pallas_call(
        _tc_mm_body,
        grid=(GRID,),
        in_specs=[
            pl.BlockSpec((BLK, D), lambda i: (i, 0)),
            pl.BlockSpec((D, D), lambda i: (0, 0)),
        ],
        out_specs=pl.BlockSpec((BLK, D), lambda i: (i, 0)),
        out_shape=jax.ShapeDtypeStruct((N, D), _f32),
    )(x, W1)


def _tc_first_body(h_ref, dp_ref, hsp_ref, dinv_ref):
    deg = 1.0 + dp_ref[0, :, 0:1] + dp_ref[1, :, 0:1]
    r0 = lax.rsqrt(deg)
    dinv = r0 * (1.5 - 0.5 * deg * r0 * r0)
    h = h_ref[...] * dinv
    hsp_ref[0] = h[:, :DH]
    hsp_ref[1] = h[:, DH:]
    dinv_ref[...] = jnp.broadcast_to(dinv, (BLK, 16))


def _pre_relu(a_ref, hsp_ref, dinv, b_ref):
    agg = jnp.concatenate([a_ref[0] + hsp_ref[0], a_ref[1] + hsp_ref[1]], axis=1)
    return jnp.maximum(agg * dinv + b_ref[...], 0.0)


def _tc_mid_body(a_ref, hsp_ref, dinv_ref, b_ref, w_ref, out_ref):
    dinv = dinv_ref[:, 0:1]
    h = _pre_relu(a_ref, hsp_ref, dinv, b_ref)
    hw = jnp.dot(h, w_ref[...], preferred_element_type=_f32) * dinv
    out_ref[0] = hw[:, :DH]
    out_ref[1] = hw[:, DH:]


def _tc_head_body(a_ref, hsp_ref, dinv_ref, b_ref, wh_ref, bh_ref, out_ref, acc_ref):
    i = pl.program_id(0)

    @pl.when(i == 0)
    def _():
        acc_ref[...] = jnp.zeros_like(acc_ref)

    dinv = dinv_ref[:, 0:1]
    h = _pre_relu(a_ref, hsp_ref, dinv, b_ref)
    acc_ref[...] += jnp.sum(h, axis=0, keepdims=True)

    @pl.when(i == GRID - 1)
    def _():
        g = acc_ref[...] * np.float32(1.0 / N)
        out_ref[...] = jnp.dot(g, wh_ref[...], preferred_element_type=_f32) + bh_ref[...]


def _tc_first(h1, degp):
    return pl.pallas_call(
        _tc_first_body,
        grid=(GRID,),
        in_specs=[
            pl.BlockSpec((BLK, D), lambda i: (i, 0)),
            pl.BlockSpec((NC, BLK, 16), lambda i: (0, i, 0)),
        ],
        out_specs=[
            pl.BlockSpec((NC, BLK, DH), lambda i: (0, i, 0)),
            pl.BlockSpec((BLK, 16), lambda i: (i, 0)),
        ],
        out_shape=[
            jax.ShapeDtypeStruct((NC, N, DH), _f32),
            jax.ShapeDtypeStruct((N, 16), _f32),
        ],
    )(h1, degp)


def _tc_mid(aggp, hsp, dinv16, b, W):
    return pl.pallas_call(
        _tc_mid_body,
        grid=(GRID,),
        in_specs=[
            pl.BlockSpec((NC, BLK, DH), lambda i: (0, i, 0)),
            pl.BlockSpec((NC, BLK, DH), lambda i: (0, i, 0)),
            pl.BlockSpec((BLK, 16), lambda i: (i, 0)),
            pl.BlockSpec((1, D), lambda i: (0, 0)),
            pl.BlockSpec((D, D), lambda i: (0, 0)),
        ],
        out_specs=pl.BlockSpec((NC, BLK, DH), lambda i: (0, i, 0)),
        out_shape=jax.ShapeDtypeStruct((NC, N, DH), _f32),
    )(aggp, hsp, dinv16, b, W)


def _tc_head(aggp, hsp, dinv16, b, Wh, bh):
    return pl.pallas_call(
        _tc_head_body,
        grid=(GRID,),
        in_specs=[
            pl.BlockSpec((NC, BLK, DH), lambda i: (0, i, 0)),
            pl.BlockSpec((NC, BLK, DH), lambda i: (0, i, 0)),
            pl.BlockSpec((BLK, 16), lambda i: (i, 0)),
            pl.BlockSpec((1, D), lambda i: (0, 0)),
            pl.BlockSpec((D, 1), lambda i: (0, 0)),
            pl.BlockSpec((1, 1), lambda i: (0, 0)),
        ],
        out_specs=pl.BlockSpec((1, 1), lambda i: (0, 0)),
        out_shape=jax.ShapeDtypeStruct((1, 1), _f32),
        scratch_shapes=[pltpu.VMEM((1, D), _f32)],
    )(aggp, hsp, dinv16, b, Wh, bh)


def kernel(x, edge_index, W1, b1, W2, b2, Wh, bh):
    ei = edge_index.astype(jnp.int32)
    src = ei[0]
    dst = ei[1]

    srcA = src.reshape(NS, NCHA, CH)
    dstA = dst.reshape(NS, NCHA, CH)
    dstD = dst.reshape(NW, NCHD, CHD)

    zD = jnp.zeros((NP, 16), _f32)
    zA = jnp.zeros((NP, DH), _f32)
    h1 = _tc_mm(x, W1)
    degp = _deg_sc(dstD, zD)
    h1sp, dinv16 = _tc_first(h1, degp)
    agg1 = _agg_sc(h1sp, srcA, dstA, zA)
    h2sp = _tc_mid(agg1, h1sp, dinv16, b1.reshape(1, D), W2)
    agg2 = _agg_sc(h2sp, srcA, dstA, zA)
    return _tc_head(agg2, h2sp, dinv16, b2.reshape(1, D), Wh, bh.reshape(1, 1))


# speedup vs baseline: 1.0510x; 1.0510x over previous
import functools

import jax
import jax.numpy as jnp
import numpy as np
from jax import lax
from jax.experimental import pallas as pl
from jax.experimental.pallas import tpu as pltpu
from jax.experimental.pallas import tpu_sc as plsc

N = 10000
E = 320000
D = 128

NC = 2
NS = 16
NW = NC * NS
DH = D // NC
CH = 128
NCHA = 160
CHD = 64
NCHD = 160
EP = NS * NCHA * CH
NBUF = 5
DBUF = 4
RPT = 632
NP = NS * RPT
BLK = 1000
GRID = N // BLK

_f32 = jnp.float32
_mesh = plsc.VectorSubcoreMesh(core_axis_name="c", subcore_axis_name="s")


@functools.partial(
    pl.kernel,
    mesh=_mesh,
    compiler_params=pltpu.CompilerParams(use_tc_tiling_on_sc=False),
    out_type=jax.ShapeDtypeStruct((NC, NP, 16), _f32),
    scratch_types=[
        pltpu.VMEM_SHARED((NP, 16), _f32),
        pltpu.VMEM((CHD, 16), _f32),
        pltpu.VMEM((NCHD, CHD), jnp.int32),
        pltpu.SemaphoreType.DMA,
        pltpu.SemaphoreType.DMA,
        pltpu.SemaphoreType.DMA,
        pltpu.SemaphoreType.DMA,
    ],
)
def _deg_sc(dst_hbm, z_hbm, out_hbm, shared, ones_v, didx, s0, s1, s2, s3):
    c = lax.axis_index("c")
    s = lax.axis_index("s")
    ssems = [s0, s1, s2, s3]

    def fill_ones(i, carry):
        ones_v[i] = jnp.full((16,), 1.0, _f32)
        return carry
    lax.fori_loop(0, CHD, fill_ones, 0)

    pltpu.sync_copy(dst_hbm.at[c * NS + s], didx)
    pltpu.sync_copy(z_hbm.at[pl.ds(s * RPT, RPT)],
                    shared.at[pl.ds(s * RPT, RPT)])
    plsc.subcore_barrier()

    def group(g, carry):
        for b in range(DBUF):
            t = g * DBUF + b

            @pl.when(t >= DBUF)
            def _():
                pltpu.make_async_copy(ones_v, shared.at[didx.at[t - DBUF]],
                                      ssems[b]).wait()
            pltpu.async_copy(ones_v, shared.at[didx.at[t]], ssems[b], add=True)
        return carry
    lax.fori_loop(0, NCHD // DBUF, group, 0)
    for b in range(DBUF):
        pltpu.make_async_copy(ones_v, shared.at[didx.at[NCHD - DBUF + b]],
                              ssems[b]).wait()

    plsc.subcore_barrier()
    pltpu.sync_copy(shared.at[pl.ds(s * RPT, RPT)],
                    out_hbm.at[c, pl.ds(s * RPT, RPT)])


@functools.partial(
    pl.kernel,
    mesh=_mesh,
    compiler_params=pltpu.CompilerParams(use_tc_tiling_on_sc=False),
    out_type=jax.ShapeDtypeStruct((NC, NP, DH), _f32),
    scratch_types=[
        pltpu.VMEM_SHARED((NP, DH), _f32),
        pltpu.VMEM((CH, DH), _f32),
        pltpu.VMEM((CH, DH), _f32),
        pltpu.VMEM((CH, DH), _f32),
        pltpu.VMEM((CH, DH), _f32),
        pltpu.VMEM((CH, DH), _f32),
        pltpu.VMEM((NCHA, CH), jnp.int32),
        pltpu.VMEM((NCHA, CH), jnp.int32),
        pltpu.SemaphoreType.DMA,
        pltpu.SemaphoreType.DMA,
        pltpu.SemaphoreType.DMA,
        pltpu.SemaphoreType.DMA,
        pltpu.SemaphoreType.DMA,
        pltpu.SemaphoreType.DMA,
        pltpu.SemaphoreType.DMA,
        pltpu.SemaphoreType.DMA,
        pltpu.SemaphoreType.DMA,
        pltpu.SemaphoreType.DMA,
    ],
)
def _agg_sc(h_hbm, src_hbm, dst_hbm, z_hbm, out_hbm, shared,
            r0, r1, r2, r3, r4, sidx, didx,
            g0, g1, g2, g3, g4,
            t0, t1, t2, t3, t4):
    c = lax.axis_index("c")
    s = lax.axis_index("s")
    rows = [r0, r1, r2, r3, r4]
    gsems = [g0, g1, g2, g3, g4]
    ssems = [t0, t1, t2, t3, t4]
    hsrc = h_hbm.at[c]

    pltpu.sync_copy(src_hbm.at[s], sidx)
    pltpu.sync_copy(dst_hbm.at[s], didx)
    pltpu.sync_copy(z_hbm.at[pl.ds(s * RPT, RPT)],
                    shared.at[pl.ds(s * RPT, RPT)])
    plsc.subcore_barrier()

    for b in range(NBUF):
        pltpu.async_copy(hsrc.at[sidx.at[b]], rows[b], gsems[b])

    def group(g, carry):
        for b in range(NBUF):
            t = g * NBUF + b
            bp = (b - 1) % NBUF
            pltpu.make_async_copy(hsrc.at[sidx.at[t]], rows[b], gsems[b]).wait()
            pltpu.async_copy(rows[b], shared.at[didx.at[t]], ssems[b], add=True)

            @pl.when(jnp.logical_and(t >= 1, t + NBUF - 1 < NCHA))
            def _():
                pltpu.make_async_copy(rows[bp], shared.at[didx.at[t - 1]],
                                      ssems[bp]).wait()
                pltpu.async_copy(hsrc.at[sidx.at[t + NBUF - 1]], rows[bp],
                                 gsems[bp])
        return carry
    lax.fori_loop(0, NCHA // NBUF, group, 0)

    for b in range(NBUF):
        t = NCHA - NBUF + b
        pltpu.make_async_copy(rows[b], shared.at[didx.at[t]], ssems[b]).wait()

    plsc.subcore_barrier()
    pltpu.sync_copy(shared.at[pl.ds(s * RPT, RPT)],
                    out_hbm.at[c, pl.ds(s * RPT, RPT)])


def _tc_mm_body(x_ref, w_ref, h_ref):
    h_ref[...] = jnp.dot(x_ref[...], w_ref[...], preferred_element_type=_f32)


def _tc_mm(x, W1):
    return pl.pallas_call(
        _tc_mm_body,
        grid=(GRID,),
        in_specs=[
            pl.BlockSpec((BLK, D), lambda i: (i, 0)),
            pl.BlockSpec((D, D), lambda i: (0, 0)),
        ],
        out_specs=pl.BlockSpec((BLK, D), lambda i: (i, 0)),
        out_shape=jax.ShapeDtypeStruct((N, D), _f32),
    )(x, W1)


def _tc_first_body(h_ref, dp_ref, hsp_ref, dinv_ref):
    deg = 1.0 + dp_ref[0, :, 0:1] + dp_ref[1, :, 0:1]
    r0 = lax.rsqrt(deg)
    dinv = r0 * (1.5 - 0.5 * deg * r0 * r0)
    h = h_ref[...] * dinv
    hsp_ref[0] = h[:, :DH]
    hsp_ref[1] = h[:, DH:]
    dinv_ref[...] = jnp.broadcast_to(dinv, (BLK, 16))


def _pre_relu(a_ref, hsp_ref, dinv, b_ref):
    agg = jnp.concatenate([a_ref[0] + hsp_ref[0], a_ref[1] + hsp_ref[1]], axis=1)
    return jnp.maximum(agg * dinv + b_ref[...], 0.0)


def _tc_mid_body(a_ref, hsp_ref, dinv_ref, b_ref, w_ref, out_ref):
    dinv = dinv_ref[:, 0:1]
    h = _pre_relu(a_ref, hsp_ref, dinv, b_ref)
    hw = jnp.dot(h, w_ref[...], preferred_element_type=_f32) * dinv
    out_ref[0] = hw[:, :DH]
    out_ref[1] = hw[:, DH:]


def _tc_head_body(a_ref, hsp_ref, dinv_ref, b_ref, wh_ref, bh_ref, out_ref, acc_ref):
    i = pl.program_id(0)

    @pl.when(i == 0)
    def _():
        acc_ref[...] = jnp.zeros_like(acc_ref)

    dinv = dinv_ref[:, 0:1]
    h = _pre_relu(a_ref, hsp_ref, dinv, b_ref)
    acc_ref[...] += jnp.sum(h, axis=0, keepdims=True)

    @pl.when(i == GRID - 1)
    def _():
        g = acc_ref[...] * np.float32(1.0 / N)
        out_ref[...] = jnp.dot(g, wh_ref[...], preferred_element_type=_f32) + bh_ref[...]


def _tc_first(h1, degp):
    return pl.pallas_call(
        _tc_first_body,
        grid=(GRID,),
        in_specs=[
            pl.BlockSpec((BLK, D), lambda i: (i, 0)),
            pl.BlockSpec((NC, BLK, 16), lambda i: (0, i, 0)),
        ],
        out_specs=[
            pl.BlockSpec((NC, BLK, DH), lambda i: (0, i, 0)),
            pl.BlockSpec((BLK, 16), lambda i: (i, 0)),
        ],
        out_shape=[
            jax.ShapeDtypeStruct((NC, N, DH), _f32),
            jax.ShapeDtypeStruct((N, 16), _f32),
        ],
    )(h1, degp)


def _tc_mid(aggp, hsp, dinv16, b, W):
    return pl.pallas_call(
        _tc_mid_body,
        grid=(GRID,),
        in_specs=[
            pl.BlockSpec((NC, BLK, DH), lambda i: (0, i, 0)),
            pl.BlockSpec((NC, BLK, DH), lambda i: (0, i, 0)),
            pl.BlockSpec((BLK, 16), lambda i: (i, 0)),
            pl.BlockSpec((1, D), lambda i: (0, 0)),
            pl.BlockSpec((D, D), lambda i: (0, 0)),
        ],
        out_specs=pl.BlockSpec((NC, BLK, DH), lambda i: (0, i, 0)),
        out_shape=jax.ShapeDtypeStruct((NC, N, DH), _f32),
    )(aggp, hsp, dinv16, b, W)


def _tc_head(aggp, hsp, dinv16, b, Wh, bh):
    return pl.pallas_call(
        _tc_head_body,
        grid=(GRID,),
        in_specs=[
            pl.BlockSpec((NC, BLK, DH), lambda i: (0, i, 0)),
            pl.BlockSpec((NC, BLK, DH), lambda i: (0, i, 0)),
            pl.BlockSpec((BLK, 16), lambda i: (i, 0)),
            pl.BlockSpec((1, D), lambda i: (0, 0)),
            pl.BlockSpec((D, 1), lambda i: (0, 0)),
            pl.BlockSpec((1, 1), lambda i: (0, 0)),
        ],
        out_specs=pl.BlockSpec((1, 1), lambda i: (0, 0)),
        out_shape=jax.ShapeDtypeStruct((1, 1), _f32),
        scratch_shapes=[pltpu.VMEM((1, D), _f32)],
    )(aggp, hsp, dinv16, b, Wh, bh)


def kernel(x, edge_index, W1, b1, W2, b2, Wh, bh):
    ei = edge_index.astype(jnp.int32)
    src = ei[0]
    dst = ei[1]

    npad = EP - E
    pad_src = (jnp.arange(npad, dtype=jnp.int32) * 37) % N
    pad_dst = N + jnp.arange(npad, dtype=jnp.int32) % (NP - N)
    src_p = jnp.concatenate([src, pad_src])
    dst_p = jnp.concatenate([dst, pad_dst])
    srcA = src_p.reshape(NS, NCHA, CH)
    dstA = dst_p.reshape(NS, NCHA, CH)
    dstD = dst_p.reshape(NW, NCHD, CHD)

    zD = jnp.zeros((NP, 16), _f32)
    zA = jnp.zeros((NP, DH), _f32)
    h1 = _tc_mm(x, W1)
    degp = _deg_sc(dstD, zD)
    h1sp, dinv16 = _tc_first(h1, degp)
    agg1 = _agg_sc(h1sp, srcA, dstA, zA)
    h2sp = _tc_mid(agg1, h1sp, dinv16, b1.reshape(1, D), W2)
    agg2 = _agg_sc(h2sp, srcA, dstA, zA)
    return _tc_head(agg2, h2sp, dinv16, b2.reshape(1, D), Wh, bh.reshape(1, 1))


# speedup vs baseline: 1.0517x; 1.0006x over previous
import functools

import jax
import jax.numpy as jnp
import numpy as np
from jax import lax
from jax.experimental import pallas as pl
from jax.experimental.pallas import tpu as pltpu
from jax.experimental.pallas import tpu_sc as plsc

N = 10000
E = 320000
D = 128

NC = 2
NS = 16
NW = NC * NS
DH = D // NC
CH = 128
NCHA = 160
CHD = 64
NCHD = 160
EP = NS * NCHA * CH
NBUF = 5
DBUF = 4
RPT = 632
NP = NS * RPT
BLK = 1000
GRID = N // BLK

_f32 = jnp.float32
_mesh = plsc.VectorSubcoreMesh(core_axis_name="c", subcore_axis_name="s")


@functools.partial(
    pl.kernel,
    mesh=_mesh,
    compiler_params=pltpu.CompilerParams(use_tc_tiling_on_sc=False),
    out_type=jax.ShapeDtypeStruct((NC, NP, 8), _f32),
    scratch_types=[
        pltpu.VMEM_SHARED((NP, 8), _f32),
        pltpu.VMEM((CHD, 8), _f32),
        pltpu.VMEM((NCHD, CHD), jnp.int32),
        pltpu.SemaphoreType.DMA,
        pltpu.SemaphoreType.DMA,
        pltpu.SemaphoreType.DMA,
        pltpu.SemaphoreType.DMA,
    ],
)
def _deg_sc(dst_hbm, z_hbm, out_hbm, shared, ones_v, didx, s0, s1, s2, s3):
    c = lax.axis_index("c")
    s = lax.axis_index("s")
    ssems = [s0, s1, s2, s3]

    def fill_ones(i, carry):
        ones_v[pl.ds(i * 2, 2)] = jnp.full((16,), 1.0, _f32).reshape(2, 8)
        return carry
    lax.fori_loop(0, CHD // 2, fill_ones, 0)

    pltpu.sync_copy(dst_hbm.at[c * NS + s], didx)
    pltpu.sync_copy(z_hbm.at[pl.ds(s * RPT, RPT)],
                    shared.at[pl.ds(s * RPT, RPT)])
    plsc.subcore_barrier()

    def group(g, carry):
        for b in range(DBUF):
            t = g * DBUF + b

            @pl.when(t >= DBUF)
            def _():
                pltpu.make_async_copy(ones_v, shared.at[didx.at[t - DBUF]],
                                      ssems[b]).wait()
            pltpu.async_copy(ones_v, shared.at[didx.at[t]], ssems[b], add=True)
        return carry
    lax.fori_loop(0, NCHD // DBUF, group, 0)
    for b in range(DBUF):
        pltpu.make_async_copy(ones_v, shared.at[didx.at[NCHD - DBUF + b]],
                              ssems[b]).wait()

    plsc.subcore_barrier()
    pltpu.sync_copy(shared.at[pl.ds(s * RPT, RPT)],
                    out_hbm.at[c, pl.ds(s * RPT, RPT)])


@functools.partial(
    pl.kernel,
    mesh=_mesh,
    compiler_params=pltpu.CompilerParams(use_tc_tiling_on_sc=False),
    out_type=jax.ShapeDtypeStruct((NC, NP, DH), _f32),
    scratch_types=[
        pltpu.VMEM_SHARED((NP, DH), _f32),
        pltpu.VMEM((CH, DH), _f32),
        pltpu.VMEM((CH, DH), _f32),
        pltpu.VMEM((CH, DH), _f32),
        pltpu.VMEM((CH, DH), _f32),
        pltpu.VMEM((CH, DH), _f32),
        pltpu.VMEM((NCHA, CH), jnp.int32),
        pltpu.VMEM((NCHA, CH), jnp.int32),
        pltpu.SemaphoreType.DMA,
        pltpu.SemaphoreType.DMA,
        pltpu.SemaphoreType.DMA,
        pltpu.SemaphoreType.DMA,
        pltpu.SemaphoreType.DMA,
        pltpu.SemaphoreType.DMA,
        pltpu.SemaphoreType.DMA,
        pltpu.SemaphoreType.DMA,
        pltpu.SemaphoreType.DMA,
        pltpu.SemaphoreType.DMA,
    ],
)
def _agg_sc(h_hbm, src_hbm, dst_hbm, z_hbm, out_hbm, shared,
            r0, r1, r2, r3, r4, sidx, didx,
            g0, g1, g2, g3, g4,
            t0, t1, t2, t3, t4):
    c = lax.axis_index("c")
    s = lax.axis_index("s")
    rows = [r0, r1, r2, r3, r4]
    gsems = [g0, g1, g2, g3, g4]
    ssems = [t0, t1, t2, t3, t4]
    hsrc = h_hbm.at[c]

    pltpu.sync_copy(src_hbm.at[s], sidx)
    pltpu.sync_copy(dst_hbm.at[s], didx)
    pltpu.sync_copy(z_hbm.at[pl.ds(s * RPT, RPT)],
                    shared.at[pl.ds(s * RPT, RPT)])
    plsc.subcore_barrier()

    for b in range(NBUF):
        pltpu.async_copy(hsrc.at[sidx.at[b]], rows[b], gsems[b])

    def group(g, carry):
        for b in range(NBUF):
            t = g * NBUF + b
            bp = (b - 1) % NBUF
            pltpu.make_async_copy(hsrc.at[sidx.at[t]], rows[b], gsems[b]).wait()
            pltpu.async_copy(rows[b], shared.at[didx.at[t]], ssems[b], add=True)

            @pl.when(jnp.logical_and(t >= 1, t + NBUF - 1 < NCHA))
            def _():
                pltpu.make_async_copy(rows[bp], shared.at[didx.at[t - 1]],
                                      ssems[bp]).wait()
                pltpu.async_copy(hsrc.at[sidx.at[t + NBUF - 1]], rows[bp],
                                 gsems[bp])
        return carry
    lax.fori_loop(0, NCHA // NBUF, group, 0)

    for b in range(NBUF):
        t = NCHA - NBUF + b
        pltpu.make_async_copy(rows[b], shared.at[didx.at[t]], ssems[b]).wait()

    plsc.subcore_barrier()
    pltpu.sync_copy(shared.at[pl.ds(s * RPT, RPT)],
                    out_hbm.at[c, pl.ds(s * RPT, RPT)])


def _tc_mm_body(x_ref, w_ref, h_ref):
    h_ref[...] = jnp.dot(x_ref[...], w_ref[...], preferred_element_type=_f32)


def _tc_mm(x, W1):
    return pl.pallas_call(
        _tc_mm_body,
        grid=(GRID,),
        in_specs=[
            pl.BlockSpec((BLK, D), lambda i: (i, 0)),
            pl.BlockSpec((D, D), lambda i: (0, 0)),
        ],
        out_specs=pl.BlockSpec((BLK, D), lambda i: (i, 0)),
        out_shape=jax.ShapeDtypeStruct((N, D), _f32),
    )(x, W1)


def _tc_first_body(h_ref, dp_ref, hsp_ref, dinv_ref):
    deg = 1.0 + dp_ref[0, :, 0:1] + dp_ref[1, :, 0:1]
    r0 = lax.rsqrt(deg)
    dinv = r0 * (1.5 - 0.5 * deg * r0 * r0)
    h = h_ref[...] * dinv
    hsp_ref[0] = h[:, :DH]
    hsp_ref[1] = h[:, DH:]
    dinv_ref[...] = jnp.broadcast_to(dinv, (BLK, 16))


def _pre_relu(a_ref, hsp_ref, dinv, b_ref):
    agg = jnp.concatenate([a_ref[0] + hsp_ref[0], a_ref[1] + hsp_ref[1]], axis=1)
    return jnp.maximum(agg * dinv + b_ref[...], 0.0)


def _tc_mid_body(a_ref, hsp_ref, dinv_ref, b_ref, w_ref, out_ref):
    dinv = dinv_ref[:, 0:1]
    h = _pre_relu(a_ref, hsp_ref, dinv, b_ref)
    hw = jnp.dot(h, w_ref[...], preferred_element_type=_f32) * dinv
    out_ref[0] = hw[:, :DH]
    out_ref[1] = hw[:, DH:]


def _tc_head_body(a_ref, hsp_ref, dinv_ref, b_ref, wh_ref, bh_ref, out_ref, acc_ref):
    i = pl.program_id(0)

    @pl.when(i == 0)
    def _():
        acc_ref[...] = jnp.zeros_like(acc_ref)

    dinv = dinv_ref[:, 0:1]
    h = _pre_relu(a_ref, hsp_ref, dinv, b_ref)
    acc_ref[...] += jnp.sum(h, axis=0, keepdims=True)

    @pl.when(i == GRID - 1)
    def _():
        g = acc_ref[...] * np.float32(1.0 / N)
        out_ref[...] = jnp.dot(g, wh_ref[...], preferred_element_type=_f32) + bh_ref[...]


def _tc_first(h1, degp):
    return pl.pallas_call(
        _tc_first_body,
        grid=(GRID,),
        in_specs=[
            pl.BlockSpec((BLK, D), lambda i: (i, 0)),
            pl.BlockSpec((NC, BLK, 8), lambda i: (0, i, 0)),
        ],
        out_specs=[
            pl.BlockSpec((NC, BLK, DH), lambda i: (0, i, 0)),
            pl.BlockSpec((BLK, 16), lambda i: (i, 0)),
        ],
        out_shape=[
            jax.ShapeDtypeStruct((NC, N, DH), _f32),
            jax.ShapeDtypeStruct((N, 16), _f32),
        ],
    )(h1, degp)


def _tc_mid(aggp, hsp, dinv16, b, W):
    return pl.pallas_call(
        _tc_mid_body,
        grid=(GRID,),
        in_specs=[
            pl.BlockSpec((NC, BLK, DH), lambda i: (0, i, 0)),
            pl.BlockSpec((NC, BLK, DH), lambda i: (0, i, 0)),
            pl.BlockSpec((BLK, 16), lambda i: (i, 0)),
            pl.BlockSpec((1, D), lambda i: (0, 0)),
            pl.BlockSpec((D, D), lambda i: (0, 0)),
        ],
        out_specs=pl.BlockSpec((NC, BLK, DH), lambda i: (0, i, 0)),
        out_shape=jax.ShapeDtypeStruct((NC, N, DH), _f32),
    )(aggp, hsp, dinv16, b, W)


def _tc_head(aggp, hsp, dinv16, b, Wh, bh):
    return pl.pallas_call(
        _tc_head_body,
        grid=(GRID,),
        in_specs=[
            pl.BlockSpec((NC, BLK, DH), lambda i: (0, i, 0)),
            pl.BlockSpec((NC, BLK, DH), lambda i: (0, i, 0)),
            pl.BlockSpec((BLK, 16), lambda i: (i, 0)),
            pl.BlockSpec((1, D), lambda i: (0, 0)),
            pl.BlockSpec((D, 1), lambda i: (0, 0)),
            pl.BlockSpec((1, 1), lambda i: (0, 0)),
        ],
        out_specs=pl.BlockSpec((1, 1), lambda i: (0, 0)),
        out_shape=jax.ShapeDtypeStruct((1, 1), _f32),
        scratch_shapes=[pltpu.VMEM((1, D), _f32)],
    )(aggp, hsp, dinv16, b, Wh, bh)


def kernel(x, edge_index, W1, b1, W2, b2, Wh, bh):
    ei = edge_index.astype(jnp.int32)
    src = ei[0]
    dst = ei[1]

    npad = EP - E
    pad_src = (jnp.arange(npad, dtype=jnp.int32) * 37) % N
    pad_dst = N + jnp.arange(npad, dtype=jnp.int32) % (NP - N)
    src_p = jnp.concatenate([src, pad_src])
    dst_p = jnp.concatenate([dst, pad_dst])
    srcA = src_p.reshape(NS, NCHA, CH)
    dstA = dst_p.reshape(NS, NCHA, CH)
    dstD = dst_p.reshape(NW, NCHD, CHD)

    zD = jnp.zeros((NP, 8), _f32)
    zA = jnp.zeros((NP, DH), _f32)
    h1 = _tc_mm(x, W1)
    degp = _deg_sc(dstD, zD)
    h1sp, dinv16 = _tc_first(h1, degp)
    agg1 = _agg_sc(h1sp, srcA, dstA, zA)
    h2sp = _tc_mid(agg1, h1sp, dinv16, b1.reshape(1, D), W2)
    agg2 = _agg_sc(h2sp, srcA, dstA, zA)
    return _tc_head(agg2, h2sp, dinv16, b2.reshape(1, D), Wh, bh.reshape(1, 1))
